# baseline (speedup 1.0000x reference)
import functools

import jax
import jax.numpy as jnp
from jax import lax
from jax.experimental import pallas as pl
from jax.experimental.pallas import tpu as pltpu
from jax.experimental.pallas import tpu_sc as plsc

N_WORDS = 1_000_000
N_CPOS = 64
BATCH = 4096
SEQ = 200
NTOK = BATCH * SEQ

_BC = 8192
_GRIDL = -(-N_WORDS // _BC)


def _lse_trans_kernel(et_ref, t_ref, lse_ref, trans_ref, m_ref, s_ref):
    step = pl.program_id(0)

    @pl.when(step == 0)
    def _init():
        m_ref[...] = jnp.full((64, 1), -jnp.inf, jnp.float32)
        s_ref[...] = jnp.zeros((64, 1), jnp.float32)

    blk = et_ref[...]
    col = step * _BC + jax.lax.broadcasted_iota(jnp.int32, (64, _BC), 1)
    valid = col < N_WORDS
    blk = jnp.where(valid, blk, -jnp.inf)
    bmax = jnp.max(blk, axis=1, keepdims=True)
    m_old = m_ref[...]
    m_new = jnp.maximum(m_old, bmax)
    e = jnp.where(valid, jnp.exp(blk - m_new), 0.0)
    s_ref[...] = (s_ref[...] * jnp.exp(m_old - m_new)
                  + jnp.sum(e, axis=1, keepdims=True))
    m_ref[...] = m_new

    @pl.when(step == pl.num_programs(0) - 1)
    def _finish():
        lse_ref[...] = m_ref[...] + jnp.log(s_ref[...])
        t = t_ref[...]
        tm = jnp.max(t, axis=1, keepdims=True)
        ts = t - tm
        trans_ref[...] = ts - jnp.log(jnp.sum(jnp.exp(ts), axis=1, keepdims=True))


def _norm_t_kernel(et_ref, lse_ref, out_ref):
    t = (et_ref[...] - lse_ref[...]).T
    out_ref[...] = jnp.concatenate([t, t], axis=1)


def _norm_table(et, lse):
    return pl.pallas_call(
        _norm_t_kernel,
        grid=(_GRIDL,),
        in_specs=[
            pl.BlockSpec((64, _BC), lambda i: (0, i)),
            pl.BlockSpec((64, 1), lambda i: (0, 0)),
        ],
        out_specs=pl.BlockSpec((_BC, 128), lambda i: (i, 0)),
        out_shape=jax.ShapeDtypeStruct((N_WORDS, 128), jnp.float32),
    )(et, lse)


def _lse_and_trans(et, t):
    return pl.pallas_call(
        _lse_trans_kernel,
        grid=(_GRIDL,),
        in_specs=[
            pl.BlockSpec((64, _BC), lambda i: (0, i)),
            pl.BlockSpec((64, 64), lambda i: (0, 0)),
        ],
        out_specs=[
            pl.BlockSpec((64, 1), lambda i: (0, 0)),
            pl.BlockSpec((64, 64), lambda i: (0, 0)),
        ],
        out_shape=[
            jax.ShapeDtypeStruct((64, 1), jnp.float32),
            jax.ShapeDtypeStruct((64, 64), jnp.float32),
        ],
        scratch_shapes=[
            pltpu.VMEM((64, 1), jnp.float32),
            pltpu.VMEM((64, 1), jnp.float32),
        ],
    )(et, t)


_NC = 2
_NS = 16
_NW = _NC * _NS
_PER_W = NTOK // _NW
_CH = 128
_NCH = _PER_W // _CH


def _sc_gather_kernel(w_hbm, m_hbm, e_hbm, out_hbm,
                      idx_v, mi_v, rows_v, out_v, sem):
    wid = lax.axis_index("s") * _NC + lax.axis_index("c")
    base = wid * _PER_W
    pltpu.sync_copy(w_hbm.at[wid], idx_v)
    pltpu.sync_copy(m_hbm.at[wid], mi_v)

    def chunk_body(c, _):
        pltpu.async_copy(e_hbm.at[idx_v.at[c]], rows_v, sem).wait()
        cbase = c * _CH

        def group_body(g, cr):
            r0 = g * 16
            miv = mi_v[pl.ds(cbase + r0, 16)]
            for j in range(16):
                r = r0 + j
                mfr = jnp.full((16,), jnp.where(miv[j] != 0, jnp.float32(0.0),
                                                jnp.float32(-jnp.inf)),
                               jnp.float32)
                out_v[r, pl.ds(0, 16)] = rows_v[r, pl.ds(0, 16)] + mfr
                out_v[r, pl.ds(16, 16)] = rows_v[r, pl.ds(16, 16)] + mfr
                out_v[r, pl.ds(32, 16)] = rows_v[r, pl.ds(32, 16)] + mfr
                out_v[r, pl.ds(48, 16)] = rows_v[r, pl.ds(48, 16)] + mfr
            return cr

        lax.fori_loop(0, _CH // 16, group_body, 0)
        pltpu.sync_copy(out_v, out_hbm.at[pl.ds(base + c * _CH, _CH)])
        return _

    lax.fori_loop(0, _NCH, chunk_body, 0)


def _sc_gather(words3, mask3, table):
    mesh = plsc.VectorSubcoreMesh(core_axis_name="c", subcore_axis_name="s")
    fn = functools.partial(
        pl.kernel,
        mesh=mesh,
        out_type=jax.ShapeDtypeStruct((NTOK, 128), jnp.float32),
        scratch_types=[
            pltpu.VMEM((_NCH, _CH), jnp.int32),
            pltpu.VMEM((_PER_W,), jnp.int32),
            pltpu.VMEM((_CH, 64), jnp.float32),
            pltpu.VMEM((_CH, 128), jnp.float32),
            pltpu.SemaphoreType.DMA,
        ],
        compiler_params=pltpu.CompilerParams(use_tc_tiling_on_sc=False),
    )(_sc_gather_kernel)
    return fn(words3, mask3, table)


def kernel(words, mask, E, T):
    et = E.T
    lse, trans = _lse_and_trans(et, T)
    table = _norm_table(et, lse).reshape(2 * N_WORDS, 64)
    words3 = (words.astype(jnp.int32) * 2).reshape(_NW, _NCH, _CH)
    mask3 = mask.astype(jnp.int32).reshape(_NW, _PER_W)
    out128 = _sc_gather(words3, mask3, table)
    emit = out128[:, :N_CPOS].reshape(BATCH, SEQ, N_CPOS)
    return emit, trans


# speedup vs baseline: 1.3895x; 1.3895x over previous
import functools

import jax
import jax.numpy as jnp
from jax import lax
from jax.experimental import pallas as pl
from jax.experimental.pallas import tpu as pltpu
from jax.experimental.pallas import tpu_sc as plsc

N_WORDS = 1_000_000
N_CPOS = 64
BATCH = 4096
SEQ = 200
NTOK = BATCH * SEQ

_BC = 8192
_GRIDL = -(-N_WORDS // _BC)


def _fused_tc_kernel(et_ref, t_ref, tab_ref, lse_ref, trans_ref, m_ref, s_ref):
    step = pl.program_id(0)

    @pl.when(step == 0)
    def _init():
        m_ref[...] = jnp.full((64, 1), -jnp.inf, jnp.float32)
        s_ref[...] = jnp.zeros((64, 1), jnp.float32)

    blk = et_ref[...]
    t = blk.T
    tab_ref[...] = jnp.concatenate([t, t], axis=1)

    col = step * _BC + jax.lax.broadcasted_iota(jnp.int32, (64, _BC), 1)
    valid = col < N_WORDS
    blkm = jnp.where(valid, blk, -jnp.inf)
    bmax = jnp.max(blkm, axis=1, keepdims=True)
    m_old = m_ref[...]
    m_new = jnp.maximum(m_old, bmax)
    e = jnp.where(valid, jnp.exp(blkm - m_new), 0.0)
    s_ref[...] = (s_ref[...] * jnp.exp(m_old - m_new)
                  + jnp.sum(e, axis=1, keepdims=True))
    m_ref[...] = m_new

    @pl.when(step == pl.num_programs(0) - 1)
    def _finish():
        lse_ref[...] = m_ref[...] + jnp.log(s_ref[...])
        t = t_ref[...]
        tm = jnp.max(t, axis=1, keepdims=True)
        ts = t - tm
        trans_ref[...] = ts - jnp.log(jnp.sum(jnp.exp(ts), axis=1, keepdims=True))


def _fused_tc(et, t):
    return pl.pallas_call(
        _fused_tc_kernel,
        grid=(_GRIDL,),
        in_specs=[
            pl.BlockSpec((64, _BC), lambda i: (0, i)),
            pl.BlockSpec((64, 64), lambda i: (0, 0)),
        ],
        out_specs=[
            pl.BlockSpec((_BC, 128), lambda i: (i, 0)),
            pl.BlockSpec((64, 1), lambda i: (0, 0)),
            pl.BlockSpec((64, 64), lambda i: (0, 0)),
        ],
        out_shape=[
            jax.ShapeDtypeStruct((N_WORDS, 128), jnp.float32),
            jax.ShapeDtypeStruct((64, 1), jnp.float32),
            jax.ShapeDtypeStruct((64, 64), jnp.float32),
        ],
        scratch_shapes=[
            pltpu.VMEM((64, 1), jnp.float32),
            pltpu.VMEM((64, 1), jnp.float32),
        ],
    )(et, t)


_NC = 2
_NS = 16
_NW = _NC * _NS
_PER_W = NTOK // _NW
_CH = 128
_NCH = _PER_W // _CH


def _sc_gather_kernel(w_hbm, m_hbm, lse_hbm, e_hbm, out_hbm,
                      idx_v, mi_v, lse_v, rows_v, sem):
    wid = lax.axis_index("s") * _NC + lax.axis_index("c")
    base = wid * _PER_W
    pltpu.sync_copy(w_hbm.at[wid], idx_v)
    pltpu.sync_copy(m_hbm.at[wid], mi_v)
    pltpu.sync_copy(lse_hbm, lse_v)
    n0 = -lse_v[pl.ds(0, 16)]
    n1 = -lse_v[pl.ds(16, 16)]
    n2 = -lse_v[pl.ds(32, 16)]
    n3 = -lse_v[pl.ds(48, 16)]

    def chunk_body(c, carry):
        pltpu.async_copy(e_hbm.at[idx_v.at[c]], rows_v, sem).wait()
        cbase = c * _CH

        def group_body(g, cr):
            a0, a1, a2, a3 = cr
            r0 = g * 16
            miv = mi_v[pl.ds(cbase + r0, 16)]
            for j in range(16):
                r = r0 + j
                mfr = jnp.full((16,), jnp.where(miv[j] != 0, jnp.float32(0.0),
                                                jnp.float32(-jnp.inf)),
                               jnp.float32)
                rows_v[r, pl.ds(0, 16)] = rows_v[r, pl.ds(0, 16)] + (a0 + mfr)
                rows_v[r, pl.ds(16, 16)] = rows_v[r, pl.ds(16, 16)] + (a1 + mfr)
                rows_v[r, pl.ds(32, 16)] = rows_v[r, pl.ds(32, 16)] + (a2 + mfr)
                rows_v[r, pl.ds(48, 16)] = rows_v[r, pl.ds(48, 16)] + (a3 + mfr)
            return cr

        carry = lax.fori_loop(0, _CH // 16, group_body, carry)
        pltpu.sync_copy(rows_v, out_hbm.at[pl.ds(base + c * _CH, _CH)])
        return carry

    lax.fori_loop(0, _NCH, chunk_body, (n0, n1, n2, n3))


def _sc_gather(words3, mask3, lse, table):
    mesh = plsc.VectorSubcoreMesh(core_axis_name="c", subcore_axis_name="s")
    fn = functools.partial(
        pl.kernel,
        mesh=mesh,
        out_type=jax.ShapeDtypeStruct((NTOK, 128), jnp.float32),
        scratch_types=[
            pltpu.VMEM((_NCH, _CH), jnp.int32),
            pltpu.VMEM((_PER_W,), jnp.int32),
            pltpu.VMEM((64,), jnp.float32),
            pltpu.VMEM((_CH, 128), jnp.float32),
            pltpu.SemaphoreType.DMA,
        ],
        compiler_params=pltpu.CompilerParams(use_tc_tiling_on_sc=False),
    )(_sc_gather_kernel)
    return fn(words3, mask3, lse, table)


def kernel(words, mask, E, T):
    table, lse, trans = _fused_tc(E.T, T)
    words3 = words.astype(jnp.int32).reshape(_NW, _NCH, _CH)
    mask3 = mask.astype(jnp.int32).reshape(_NW, _PER_W)
    out128 = _sc_gather(words3, mask3, lse.reshape(64), table)
    emit = out128[:, :N_CPOS].reshape(BATCH, SEQ, N_CPOS)
    return emit, trans


# speedup vs baseline: 1.7269x; 1.2428x over previous
import functools

import jax
import jax.numpy as jnp
from jax import lax
from jax.experimental import pallas as pl
from jax.experimental.pallas import tpu as pltpu
from jax.experimental.pallas import tpu_sc as plsc

N_WORDS = 1_000_000
N_CPOS = 64
BATCH = 4096
SEQ = 200
NTOK = BATCH * SEQ

_BC = 8192
_GRIDL = -(-N_WORDS // _BC)


def _fused_tc_kernel(et_ref, t_ref, tab_ref, lse_ref, trans_ref, m_ref, s_ref):
    step = pl.program_id(0)

    @pl.when(step == 0)
    def _init():
        m_ref[...] = jnp.full((64, 1), -jnp.inf, jnp.float32)
        s_ref[...] = jnp.zeros((64, 1), jnp.float32)

    blk = et_ref[...]
    t = blk.T
    tab_ref[...] = jnp.concatenate([t, t], axis=1)

    col = step * _BC + jax.lax.broadcasted_iota(jnp.int32, (64, _BC), 1)
    valid = col < N_WORDS
    blkm = jnp.where(valid, blk, -jnp.inf)
    bmax = jnp.max(blkm, axis=1, keepdims=True)
    m_old = m_ref[...]
    m_new = jnp.maximum(m_old, bmax)
    e = jnp.where(valid, jnp.exp(blkm - m_new), 0.0)
    s_ref[...] = (s_ref[...] * jnp.exp(m_old - m_new)
                  + jnp.sum(e, axis=1, keepdims=True))
    m_ref[...] = m_new

    @pl.when(step == pl.num_programs(0) - 1)
    def _finish():
        lse_ref[...] = m_ref[...] + jnp.log(s_ref[...])
        t = t_ref[...]
        tm = jnp.max(t, axis=1, keepdims=True)
        ts = t - tm
        trans_ref[...] = ts - jnp.log(jnp.sum(jnp.exp(ts), axis=1, keepdims=True))


def _fused_tc(et, t):
    return pl.pallas_call(
        _fused_tc_kernel,
        grid=(_GRIDL,),
        in_specs=[
            pl.BlockSpec((64, _BC), lambda i: (0, i)),
            pl.BlockSpec((64, 64), lambda i: (0, 0)),
        ],
        out_specs=[
            pl.BlockSpec((_BC, 128), lambda i: (i, 0)),
            pl.BlockSpec((64, 1), lambda i: (0, 0)),
            pl.BlockSpec((64, 64), lambda i: (0, 0)),
        ],
        out_shape=[
            jax.ShapeDtypeStruct((N_WORDS, 128), jnp.float32),
            jax.ShapeDtypeStruct((64, 1), jnp.float32),
            jax.ShapeDtypeStruct((64, 64), jnp.float32),
        ],
        scratch_shapes=[
            pltpu.VMEM((64, 1), jnp.float32),
            pltpu.VMEM((64, 1), jnp.float32),
        ],
    )(et, t)


_NC = 2
_NS = 16
_NW = _NC * _NS
_PER_W = NTOK // _NW
_CH = 128
_NCH = _PER_W // _CH


def _sc_gather_kernel(w_hbm, m_hbm, lse_hbm, e_hbm, out_hbm,
                      idx_v, mi_v, lse_v, rows_v, gsem, osem):
    wid = lax.axis_index("s") * _NC + lax.axis_index("c")
    base = wid * _PER_W
    pltpu.sync_copy(w_hbm.at[wid], idx_v)
    pltpu.sync_copy(m_hbm.at[wid], mi_v)
    pltpu.sync_copy(lse_hbm, lse_v)
    n0 = -lse_v[pl.ds(0, 16)]
    n1 = -lse_v[pl.ds(16, 16)]
    n2 = -lse_v[pl.ds(32, 16)]
    n3 = -lse_v[pl.ds(48, 16)]

    def compute(c, b, carry):
        cbase = c * _CH

        def group_body(g, cr):
            a0, a1, a2, a3 = cr
            r0 = g * 16
            miv = mi_v[pl.ds(cbase + r0, 16)]
            for j in range(16):
                r = r0 + j
                mfr = jnp.full((16,), jnp.where(miv[j] != 0, jnp.float32(0.0),
                                                jnp.float32(-jnp.inf)),
                               jnp.float32)
                rows_v[b, r, pl.ds(0, 16)] = rows_v[b, r, pl.ds(0, 16)] + (a0 + mfr)
                rows_v[b, r, pl.ds(16, 16)] = rows_v[b, r, pl.ds(16, 16)] + (a1 + mfr)
                rows_v[b, r, pl.ds(32, 16)] = rows_v[b, r, pl.ds(32, 16)] + (a2 + mfr)
                rows_v[b, r, pl.ds(48, 16)] = rows_v[b, r, pl.ds(48, 16)] + (a3 + mfr)
            return cr

        return lax.fori_loop(0, _CH // 16, group_body, carry)

    def out_slot(c):
        return out_hbm.at[pl.ds(base + c * _CH, _CH)]

    pltpu.async_copy(e_hbm.at[idx_v.at[0]], rows_v.at[0], gsem.at[0])

    def outer(i, carry):
        c0 = i * 2
        for b in range(2):
            c = c0 + b
            nxt = c + 1

            @pl.when(nxt < _NCH)
            def _():
                @pl.when(c >= 1)
                def _():
                    pltpu.make_async_copy(rows_v.at[1 - b], out_slot(c - 1),
                                          osem.at[1 - b]).wait()
                pltpu.async_copy(e_hbm.at[idx_v.at[nxt]], rows_v.at[1 - b],
                                 gsem.at[1 - b])

            pltpu.make_async_copy(e_hbm.at[idx_v.at[c]], rows_v.at[b],
                                  gsem.at[b]).wait()
            carry = compute(c, b, carry)
            pltpu.async_copy(rows_v.at[b], out_slot(c), osem.at[b])
        return carry

    lax.fori_loop(0, _NCH // 2, outer, (n0, n1, n2, n3))
    pltpu.make_async_copy(rows_v.at[0], out_slot(_NCH - 2), osem.at[0]).wait()
    pltpu.make_async_copy(rows_v.at[1], out_slot(_NCH - 1), osem.at[1]).wait()


def _sc_gather(words3, mask3, lse, table):
    mesh = plsc.VectorSubcoreMesh(core_axis_name="c", subcore_axis_name="s")
    fn = functools.partial(
        pl.kernel,
        mesh=mesh,
        out_type=jax.ShapeDtypeStruct((NTOK, 128), jnp.float32),
        scratch_types=[
            pltpu.VMEM((_NCH, _CH), jnp.int32),
            pltpu.VMEM((_PER_W,), jnp.int32),
            pltpu.VMEM((64,), jnp.float32),
            pltpu.VMEM((2, _CH, 128), jnp.float32),
            pltpu.SemaphoreType.DMA((2,)),
            pltpu.SemaphoreType.DMA((2,)),
        ],
        compiler_params=pltpu.CompilerParams(use_tc_tiling_on_sc=False),
    )(_sc_gather_kernel)
    return fn(words3, mask3, lse, table)


def kernel(words, mask, E, T):
    table, lse, trans = _fused_tc(E.T, T)
    words3 = words.astype(jnp.int32).reshape(_NW, _NCH, _CH)
    mask3 = mask.astype(jnp.int32).reshape(_NW, _PER_W)
    out128 = _sc_gather(words3, mask3, lse.reshape(64), table)
    emit = out128[:, :N_CPOS].reshape(BATCH, SEQ, N_CPOS)
    return emit, trans
